# baseline (speedup 1.0000x reference)
import functools
import jax
import jax.numpy as jnp
from jax import lax
from jax.experimental import pallas as pl
from jax.experimental.pallas import tpu as pltpu, tpu_sc as plsc

N = 10000
D = 128
N_PAD = 10240
SINK = 10200
K = 128
NW = 32
ROWS_PER_TILE = N_PAD // 16
BLK = 1024
GRID = N_PAD // BLK

E_LOOP_PAD = 16384
E_DEP_PAD = 163840


CH_LOOP = E_LOOP_PAD // (NW * K)
CH_DEP = E_DEP_PAD // (NW * K)
CH_TOT = CH_LOOP + 2 * CH_DEP
REL_OFF = (2 * CH_DEP, 0, CH_DEP)
REL_CH = (CH_LOOP, CH_DEP, CH_DEP)
NB = 2
NBD = 4


def _pad_edges(ei, e_pad):
    e = ei.shape[1]
    pad = jnp.full((e_pad - e,), SINK, dtype=jnp.int32)
    src = jnp.concatenate([ei[0].astype(jnp.int32), pad])
    dst = jnp.concatenate([ei[1].astype(jnp.int32), pad])
    return src, dst


def _stage_edges(edge_index_loop, edge_index_dep, edge_index_rdep):
    sl, dl = _pad_edges(edge_index_loop, E_LOOP_PAD)
    sd, dd = _pad_edges(edge_index_dep, E_DEP_PAD)
    sr, dr = _pad_edges(edge_index_rdep, E_DEP_PAD)
    def cat(dep, rdep, loop):
        return jnp.concatenate(
            [dep.reshape(NW, CH_DEP, K), rdep.reshape(NW, CH_DEP, K),
             loop.reshape(NW, CH_LOOP, K)], axis=1)
    return cat(sd, sr, sl), cat(dd, dr, dl)


def _deg_kernel(src_all, dst_all, out_hbm, idx_v, ones_v,
                deg_sh, s0, s1, s2, s3):
    cid = lax.axis_index("c")
    sid = lax.axis_index("s")
    wid = cid * 16 + sid
    row0 = sid * ROWS_PER_TILE
    sems = (s0, s1, s2, s3)

    z16 = jnp.zeros((16,), jnp.float32)

    def zinit(i, _):
        for j in range(8):
            ones_v[i, pl.ds(j * 16, 16)] = z16
        return 0
    lax.fori_loop(0, K, zinit, 0, unroll=False)

    for kk in range(ROWS_PER_TILE // K):
        pltpu.sync_copy(ones_v, deg_sh.at[pl.ds(row0 + kk * K, K)])
    plsc.subcore_barrier()

    streams = [(src_all, REL_OFF[0], CH_LOOP), (dst_all, REL_OFF[0], CH_LOOP),
               (src_all, REL_OFF[1], CH_DEP), (dst_all, REL_OFF[1], CH_DEP),
               (src_all, REL_OFF[2], CH_DEP), (dst_all, REL_OFF[2], CH_DEP)]
    for q, (arr, roff, cpw) in enumerate(streams):
        eq = jnp.where(lax.iota(jnp.int32, 16) == q, 1.0, 0.0).astype(jnp.float32)

        def init_body(i, _):
            ones_v[i, pl.ds(0, 16)] = eq
            return 0
        lax.fori_loop(0, K, init_body, 0, unroll=False)

        pltpu.sync_copy(arr.at[wid, pl.ds(roff, cpw)], idx_v.at[pl.ds(0, cpw)])

        for j in range(cpw):
            b = j % NBD
            if j >= NBD:
                pltpu.make_async_copy(ones_v, deg_sh.at[pl.ds(0, K)],
                                      sems[b]).wait()
            pltpu.async_copy(ones_v, deg_sh.at[idx_v.at[j]],
                             sems[b], add=True)
        for b in range(min(NBD, cpw)):
            pltpu.make_async_copy(ones_v, deg_sh.at[pl.ds(0, K)],
                                  sems[b]).wait()
    plsc.subcore_barrier()

    for kk in range(ROWS_PER_TILE // K):
        r0 = row0 + kk * K
        pltpu.sync_copy(deg_sh.at[pl.ds(r0, K)], ones_v)
        pltpu.sync_copy(ones_v, out_hbm.at[cid, pl.ds(r0, K)])


def _run_deg(src_all, dst_all):
    k = pl.kernel(
        _deg_kernel,
        out_type=jax.ShapeDtypeStruct((2, N_PAD, D), jnp.float32),
        mesh=plsc.VectorSubcoreMesh(core_axis_name="c", subcore_axis_name="s"),
        scratch_types=[
            pltpu.VMEM((CH_DEP, K), jnp.int32),
            pltpu.VMEM((K, D), jnp.float32),
            pltpu.VMEM_SHARED((N_PAD, D), jnp.float32),
            pltpu.SemaphoreType.DMA,
            pltpu.SemaphoreType.DMA,
            pltpu.SemaphoreType.DMA,
            pltpu.SemaphoreType.DMA,
        ],
    )
    return k(src_all, dst_all)


def _agg_kernel(yl, yd, yr, src_all, dst_all, out_hbm,
                idxs_v, idxd_v, rb0, rb1, acc_sh,
                g0, g1, s0, s1):
    cid = lax.axis_index("c")
    sid = lax.axis_index("s")
    wid = cid * 16 + sid
    row0 = sid * ROWS_PER_TILE
    rings = (rb0, rb1)
    gsems = (g0, g1)
    ssems = (s0, s1)

    z16 = jnp.zeros((16,), jnp.float32)

    rels = [(yl, REL_OFF[0], REL_CH[0]), (yd, REL_OFF[1], REL_CH[1]),
            (yr, REL_OFF[2], REL_CH[2])]
    for r, (ytab, roff, cpw) in enumerate(rels):
        def zinit(i, _):
            for j in range(8):
                rb0[i, pl.ds(j * 16, 16)] = z16
            return 0
        lax.fori_loop(0, K, zinit, 0, unroll=False)
        for kk in range(ROWS_PER_TILE // K):
            pltpu.sync_copy(rb0, acc_sh.at[pl.ds(row0 + kk * K, K)])

        pltpu.sync_copy(src_all.at[wid, pl.ds(roff, cpw)],
                        idxs_v.at[pl.ds(0, cpw)])
        pltpu.sync_copy(dst_all.at[wid, pl.ds(roff, cpw)],
                        idxd_v.at[pl.ds(0, cpw)])
        plsc.subcore_barrier()

        nb = min(NB, cpw)
        for b in range(nb):
            pltpu.async_copy(ytab.at[idxs_v.at[b]], rings[b], gsems[b])
        for g in range(cpw // nb):
            for b in range(nb):
                j = g * nb + b
                pltpu.make_async_copy(ytab.at[pl.ds(0, K)], rings[b],
                                      gsems[b]).wait()
                pltpu.async_copy(rings[b], acc_sh.at[idxd_v.at[j]],
                                 ssems[b], add=True)
            for b in range(nb):
                j2 = (g + 1) * nb + b
                if j2 < cpw:
                    pltpu.make_async_copy(rings[b], acc_sh.at[pl.ds(0, K)],
                                          ssems[b]).wait()
                    pltpu.async_copy(ytab.at[idxs_v.at[j2]], rings[b],
                                     gsems[b])
        for b in range(nb):
            pltpu.make_async_copy(rings[b], acc_sh.at[pl.ds(0, K)],
                                  ssems[b]).wait()
        plsc.subcore_barrier()

        for kk in range(ROWS_PER_TILE // K):
            r0 = row0 + kk * K
            pltpu.sync_copy(acc_sh.at[pl.ds(r0, K)], rb0)
            pltpu.sync_copy(rb0, out_hbm.at[cid, r, pl.ds(r0, K)])


def _run_agg(yl, yd, yr, src_all, dst_all):
    k = pl.kernel(
        _agg_kernel,
        out_type=jax.ShapeDtypeStruct((2, 3, N_PAD, D), jnp.float32),
        mesh=plsc.VectorSubcoreMesh(core_axis_name="c", subcore_axis_name="s"),
        scratch_types=[
            pltpu.VMEM((CH_DEP, K), jnp.int32),
            pltpu.VMEM((CH_DEP, K), jnp.int32),
            pltpu.VMEM((K, D), jnp.float32),
            pltpu.VMEM((K, D), jnp.float32),
            pltpu.VMEM_SHARED((N_PAD, D), jnp.float32),
            pltpu.SemaphoreType.DMA,
            pltpu.SemaphoreType.DMA,
            pltpu.SemaphoreType.DMA,
            pltpu.SemaphoreType.DMA,
        ],
    )
    return k(yl, yd, yr, src_all, dst_all)


def _scale_kernel(degp_ref, out_ref):
    p = degp_ref[...]
    deg = (p[0] + p[1])[:, 0:8]
    out_ref[...] = lax.rsqrt(jnp.maximum(deg, 1.0))


def _run_scale(degp):
    return pl.pallas_call(
        _scale_kernel,
        grid=(GRID,),
        in_specs=[pl.BlockSpec((2, BLK, D), lambda i: (0, i, 0))],
        out_specs=pl.BlockSpec((BLK, 8), lambda i: (i, 0)),
        out_shape=jax.ShapeDtypeStruct((N_PAD, 8), jnp.float32),
    )(degp)


def _proj0_kernel(x_ref, sc_ref, wl_ref, wd_ref, wr_ref, yl_ref, yd_ref, yr_ref):
    x = x_ref[...]
    s = sc_ref[...]
    for w_ref, y_ref, col in ((wl_ref, yl_ref, 0), (wd_ref, yd_ref, 2),
                              (wr_ref, yr_ref, 4)):
        xs = x * s[:, col][:, None]
        y_ref[...] = jnp.dot(xs, w_ref[...],
                             preferred_element_type=jnp.float32,
                             precision=lax.Precision.HIGHEST)


def _run_proj0(x_pad, scales, w0l, w0d, w0r):
    row_spec = pl.BlockSpec((BLK, D), lambda i: (i, 0))
    return pl.pallas_call(
        _proj0_kernel,
        grid=(GRID,),
        in_specs=[row_spec,
                  pl.BlockSpec((BLK, 8), lambda i: (i, 0)),
                  pl.BlockSpec((D, D), lambda i: (0, 0)),
                  pl.BlockSpec((D, D), lambda i: (0, 0)),
                  pl.BlockSpec((D, D), lambda i: (0, 0))],
        out_specs=[row_spec, row_spec, row_spec],
        out_shape=[jax.ShapeDtypeStruct((N_PAD, D), jnp.float32)] * 3,
    )(x_pad, scales, w0l, w0d, w0r)


def _combine_proj_kernel(p_ref, sc_ref, bsum_ref, wl_ref, wd_ref, wr_ref,
                         yl_ref, yd_ref, yr_ref):
    i = pl.program_id(0)
    s = sc_ref[...]
    p = p_ref[...]
    acc = (p[0, 0] + p[1, 0]) * s[:, 1][:, None]
    acc += (p[0, 1] + p[1, 1]) * s[:, 3][:, None]
    acc += (p[0, 2] + p[1, 2]) * s[:, 5][:, None]
    acc += bsum_ref[...]
    rows = i * BLK + lax.broadcasted_iota(jnp.int32, (BLK, 1), 0)
    h = jnp.where(rows < N, jnp.maximum(acc, 0.0), 0.0)
    for w_ref, y_ref, col in ((wl_ref, yl_ref, 0), (wd_ref, yd_ref, 2),
                              (wr_ref, yr_ref, 4)):
        hs = h * s[:, col][:, None]
        y_ref[...] = jnp.dot(hs, w_ref[...],
                             preferred_element_type=jnp.float32,
                             precision=lax.Precision.HIGHEST)


def _run_combine_proj(p, scales, bsum, w1l, w1d, w1r):
    row_spec = pl.BlockSpec((BLK, D), lambda i: (i, 0))
    return pl.pallas_call(
        _combine_proj_kernel,
        grid=(GRID,),
        in_specs=[pl.BlockSpec((2, 3, BLK, D), lambda i: (0, 0, i, 0)),
                  pl.BlockSpec((BLK, 8), lambda i: (i, 0)),
                  pl.BlockSpec((1, D), lambda i: (0, 0)),
                  pl.BlockSpec((D, D), lambda i: (0, 0)),
                  pl.BlockSpec((D, D), lambda i: (0, 0)),
                  pl.BlockSpec((D, D), lambda i: (0, 0))],
        out_specs=[row_spec, row_spec, row_spec],
        out_shape=[jax.ShapeDtypeStruct((N_PAD, D), jnp.float32)] * 3,
    )(p, scales, bsum, w1l, w1d, w1r)


def _readout_kernel(p_ref, sc_ref, bsum_ref, out_ref):
    i = pl.program_id(0)
    s = sc_ref[...]
    p = p_ref[...]
    acc = (p[0, 0] + p[1, 0]) * s[:, 1][:, None]
    acc += (p[0, 1] + p[1, 1]) * s[:, 3][:, None]
    acc += (p[0, 2] + p[1, 2]) * s[:, 5][:, None]
    acc += bsum_ref[...]
    rows = i * BLK + lax.broadcasted_iota(jnp.int32, (BLK, 1), 0)
    h = jnp.where(rows < N, jnp.maximum(acc, 0.0), 0.0)
    part = jnp.sum(h, axis=0, keepdims=True) * (1.0 / N)

    @pl.when(i == 0)
    def _():
        out_ref[...] = part

    @pl.when(i > 0)
    def _():
        out_ref[...] += part


def _run_readout(p, scales, bsum):
    return pl.pallas_call(
        _readout_kernel,
        grid=(GRID,),
        in_specs=[pl.BlockSpec((2, 3, BLK, D), lambda i: (0, 0, i, 0)),
                  pl.BlockSpec((BLK, 8), lambda i: (i, 0)),
                  pl.BlockSpec((1, D), lambda i: (0, 0))],
        out_specs=pl.BlockSpec((1, D), lambda i: (0, 0)),
        out_shape=jax.ShapeDtypeStruct((1, D), jnp.float32),
    )(p, scales, bsum)


def kernel(x, edge_index_loop, edge_index_dep, edge_index_rdep,
           W0_loop, b0_loop, W0_dep, b0_dep, W0_rdep, b0_rdep,
           W1_loop, b1_loop, W1_dep, b1_dep, W1_rdep, b1_rdep):
    src_all, dst_all = _stage_edges(edge_index_loop, edge_index_dep,
                                    edge_index_rdep)
    x_pad = jnp.pad(x, ((0, N_PAD - N), (0, 0)))

    degp = _run_deg(src_all, dst_all)
    scales = _run_scale(degp)

    b0sum = (b0_loop + b0_dep + b0_rdep).reshape(1, D)
    b1sum = (b1_loop + b1_dep + b1_rdep).reshape(1, D)

    y0l, y0d, y0r = _run_proj0(x_pad, scales, W0_loop, W0_dep, W0_rdep)
    p0 = _run_agg(y0l, y0d, y0r, src_all, dst_all)
    y1l, y1d, y1r = _run_combine_proj(p0, scales, b0sum, W1_loop, W1_dep, W1_rdep)
    p1 = _run_agg(y1l, y1d, y1r, src_all, dst_all)
    return _run_readout(p1, scales, b1sum)


# speedup vs baseline: 1.4480x; 1.4480x over previous
import functools
import jax
import jax.numpy as jnp
from jax import lax
from jax.experimental import pallas as pl
from jax.experimental.pallas import tpu as pltpu, tpu_sc as plsc

N = 10000
D = 128
N_PAD = 10240
SINK = 10200
K = 128
NW = 32
ROWS_PER_TILE = N_PAD // 16
BLK = 1024
GRID = N_PAD // BLK

E_LOOP_PAD = 16384
E_DEP_PAD = 163840


CH_LOOP = E_LOOP_PAD // (NW * K)
CH_DEP = E_DEP_PAD // (NW * K)
CH_TOT = CH_LOOP + 2 * CH_DEP
REL_OFF = (2 * CH_DEP, 0, CH_DEP)
REL_CH = (CH_LOOP, CH_DEP, CH_DEP)
NB = 2
NBD = 4


def _pad_edges(ei, e_pad):
    e = ei.shape[1]
    pad = jnp.full((e_pad - e,), SINK, dtype=jnp.int32)
    src = jnp.concatenate([ei[0].astype(jnp.int32), pad])
    dst = jnp.concatenate([ei[1].astype(jnp.int32), pad])
    return src, dst


def _stage_edges(edge_index_loop, edge_index_dep, edge_index_rdep):
    sl, dl = _pad_edges(edge_index_loop, E_LOOP_PAD)
    sd, dd = _pad_edges(edge_index_dep, E_DEP_PAD)
    sr, dr = _pad_edges(edge_index_rdep, E_DEP_PAD)
    def cat(dep, rdep, loop):
        return jnp.concatenate(
            [dep.reshape(NW, CH_DEP, K), rdep.reshape(NW, CH_DEP, K),
             loop.reshape(NW, CH_LOOP, K)], axis=1)
    return cat(sd, sr, sl), cat(dd, dr, dl)


def _deg_kernel(src_all, dst_all, out_hbm, idx_v, ones_v,
                deg_sh, s0, s1, s2, s3):
    cid = lax.axis_index("c")
    sid = lax.axis_index("s")
    wid = cid * 16 + sid
    row0 = sid * ROWS_PER_TILE
    sems = (s0, s1, s2, s3)

    z16 = jnp.zeros((16,), jnp.float32)

    def zinit(i, _):
        for j in range(8):
            ones_v[i, pl.ds(j * 16, 16)] = z16
        return 0
    lax.fori_loop(0, K, zinit, 0, unroll=False)

    for kk in range(ROWS_PER_TILE // K):
        pltpu.sync_copy(ones_v, deg_sh.at[pl.ds(row0 + kk * K, K)])
    plsc.subcore_barrier()

    streams = [(src_all, REL_OFF[0], CH_LOOP), (dst_all, REL_OFF[0], CH_LOOP),
               (src_all, REL_OFF[1], CH_DEP), (dst_all, REL_OFF[1], CH_DEP),
               (src_all, REL_OFF[2], CH_DEP), (dst_all, REL_OFF[2], CH_DEP)]
    for q, (arr, roff, cpw) in enumerate(streams):
        eq = jnp.where(lax.iota(jnp.int32, 16) == q, 1.0, 0.0).astype(jnp.float32)

        def init_body(i, _):
            ones_v[i, pl.ds(0, 16)] = eq
            return 0
        lax.fori_loop(0, K, init_body, 0, unroll=False)

        pltpu.sync_copy(arr.at[wid, pl.ds(roff, cpw)], idx_v.at[pl.ds(0, cpw)])

        for j in range(cpw):
            b = j % NBD
            if j >= NBD:
                pltpu.make_async_copy(ones_v, deg_sh.at[pl.ds(0, K)],
                                      sems[b]).wait()
            pltpu.async_copy(ones_v, deg_sh.at[idx_v.at[j]],
                             sems[b], add=True)
        for b in range(min(NBD, cpw)):
            pltpu.make_async_copy(ones_v, deg_sh.at[pl.ds(0, K)],
                                  sems[b]).wait()
    plsc.subcore_barrier()

    for kk in range(ROWS_PER_TILE // K):
        r0 = row0 + kk * K
        pltpu.sync_copy(deg_sh.at[pl.ds(r0, K)], ones_v)
        pltpu.sync_copy(ones_v, out_hbm.at[cid, pl.ds(r0, K)])


def _run_deg(src_all, dst_all):
    k = pl.kernel(
        _deg_kernel,
        out_type=jax.ShapeDtypeStruct((2, N_PAD, D), jnp.float32),
        mesh=plsc.VectorSubcoreMesh(core_axis_name="c", subcore_axis_name="s"),
        scratch_types=[
            pltpu.VMEM((CH_DEP, K), jnp.int32),
            pltpu.VMEM((K, D), jnp.float32),
            pltpu.VMEM_SHARED((N_PAD, D), jnp.float32),
            pltpu.SemaphoreType.DMA,
            pltpu.SemaphoreType.DMA,
            pltpu.SemaphoreType.DMA,
            pltpu.SemaphoreType.DMA,
        ],
    )
    return k(src_all, dst_all)


FAST_CID = 0
FCH = (8, 64, 56)
SCH = (0, 16, 24)
CH_ROW = 136
REL_OFF_A = (128, 0, 64)
SEG = 32


def _stage_edges_asym(edge_index_loop, edge_index_dep, edge_index_rdep):
    sl, dl = _pad_edges(edge_index_loop, E_LOOP_PAD)
    sd, dd = _pad_edges(edge_index_dep, E_DEP_PAD)
    sr, dr = _pad_edges(edge_index_rdep, E_DEP_PAD)

    def split(arr, fc, sc, slots):
        c = arr.reshape(-1, K)
        fast = c[:16 * fc].reshape(16, fc, K)
        slow = c[16 * fc:].reshape(16, sc, K) if sc else jnp.zeros(
            (16, 0, K), jnp.int32)
        fast = jnp.pad(fast, ((0, 0), (0, slots - fc), (0, 0)))
        slow = jnp.pad(slow, ((0, 0), (0, slots - sc), (0, 0)))
        pair = (fast, slow) if FAST_CID == 0 else (slow, fast)
        return jnp.concatenate(pair, axis=0)

    def lay(lp, dp, rd):
        return jnp.concatenate(
            [split(dp, FCH[1], SCH[1], 64), split(rd, FCH[2], SCH[2], 64),
             split(lp, FCH[0], SCH[0], 8)], axis=1)
    return lay(sl, sd, sr), lay(dl, dd, dr)


def _agg_kernel(yl, yd, yr, src_a, dst_a, out_hbm,
                idxs_v, idxd_v, rb0, rb1, acc_sh, g0, g1, s0, s1):
    cid = lax.axis_index("c")
    sid = lax.axis_index("s")
    wid = cid * 16 + sid
    row0 = sid * ROWS_PER_TILE
    rings = (rb0, rb1)
    gsems = (g0, g1)
    ssems = (s0, s1)

    z16 = jnp.zeros((16,), jnp.float32)

    def chunk(ytab, j, b):
        pltpu.async_copy(ytab.at[idxs_v.at[j]], rings[b], gsems[b])
        pltpu.make_async_copy(ytab.at[pl.ds(0, K)], rings[b], gsems[b]).wait()
        pltpu.async_copy(rings[b], acc_sh.at[idxd_v.at[j]], ssems[b], add=True)

    def pipeline(ytab, roff, cpw):
        for seg0 in range(0, cpw, SEG):
            seg = min(SEG, cpw - seg0)
            pltpu.sync_copy(src_a.at[wid, pl.ds(roff + seg0, seg)],
                            idxs_v.at[pl.ds(0, seg)])
            pltpu.sync_copy(dst_a.at[wid, pl.ds(roff + seg0, seg)],
                            idxd_v.at[pl.ds(0, seg)])
            for b in range(2):
                chunk(ytab, b, b)

            def grp(g, _):
                for b in range(2):
                    pltpu.make_async_copy(rings[b], acc_sh.at[pl.ds(0, K)],
                                          ssems[b]).wait()
                    chunk(ytab, g * 2 + b, b)
                return 0
            lax.fori_loop(1, seg // 2, grp, 0, unroll=False)
            for b in range(2):
                pltpu.make_async_copy(rings[b], acc_sh.at[pl.ds(0, K)],
                                      ssems[b]).wait()

    rels = [(yl, 0), (yd, 1), (yr, 2)]
    for r, (ytab, ri) in enumerate(rels):
        def zinit(i, _):
            for j in range(8):
                rb0[i, pl.ds(j * 16, 16)] = z16
            return 0
        lax.fori_loop(0, K, zinit, 0, unroll=False)
        for kk in range(ROWS_PER_TILE // K):
            pltpu.sync_copy(rb0, acc_sh.at[pl.ds(row0 + kk * K, K)])
        plsc.subcore_barrier()

        roff = REL_OFF_A[ri]
        if SCH[ri]:
            pipeline(ytab, roff, SCH[ri])
        if FCH[ri] > SCH[ri]:
            @pl.when(cid == FAST_CID)
            def _():
                pipeline(ytab, roff + SCH[ri], FCH[ri] - SCH[ri])
        plsc.subcore_barrier()

        for kk in range(ROWS_PER_TILE // K):
            r0 = row0 + kk * K
            pltpu.sync_copy(acc_sh.at[pl.ds(r0, K)], rb0)
            pltpu.sync_copy(rb0, out_hbm.at[cid, r, pl.ds(r0, K)])


def _run_agg(yl, yd, yr, src_a, dst_a):
    k = pl.kernel(
        _agg_kernel,
        out_type=jax.ShapeDtypeStruct((2, 3, N_PAD, D), jnp.float32),
        mesh=plsc.VectorSubcoreMesh(core_axis_name="c", subcore_axis_name="s"),
        scratch_types=[
            pltpu.VMEM((SEG, K), jnp.int32),
            pltpu.VMEM((SEG, K), jnp.int32),
            pltpu.VMEM((K, D), jnp.float32),
            pltpu.VMEM((K, D), jnp.float32),
            pltpu.VMEM_SHARED((N_PAD, D), jnp.float32),
            pltpu.SemaphoreType.DMA,
            pltpu.SemaphoreType.DMA,
            pltpu.SemaphoreType.DMA,
            pltpu.SemaphoreType.DMA,
        ],
    )
    return k(yl, yd, yr, src_a, dst_a)


def _scale_kernel(degp_ref, out_ref):
    p = degp_ref[...]
    deg = (p[0] + p[1])[:, 0:8]
    out_ref[...] = lax.rsqrt(jnp.maximum(deg, 1.0))


def _run_scale(degp):
    return pl.pallas_call(
        _scale_kernel,
        grid=(GRID,),
        in_specs=[pl.BlockSpec((2, BLK, D), lambda i: (0, i, 0))],
        out_specs=pl.BlockSpec((BLK, 8), lambda i: (i, 0)),
        out_shape=jax.ShapeDtypeStruct((N_PAD, 8), jnp.float32),
    )(degp)


def _proj0_kernel(x_ref, sc_ref, wl_ref, wd_ref, wr_ref, yl_ref, yd_ref, yr_ref):
    x = x_ref[...]
    s = sc_ref[...]
    for w_ref, y_ref, col in ((wl_ref, yl_ref, 0), (wd_ref, yd_ref, 2),
                              (wr_ref, yr_ref, 4)):
        xs = x * s[:, col][:, None]
        y_ref[...] = jnp.dot(xs, w_ref[...],
                             preferred_element_type=jnp.float32,
                             precision=lax.Precision.HIGHEST)


def _run_proj0(x_pad, scales, w0l, w0d, w0r):
    row_spec = pl.BlockSpec((BLK, D), lambda i: (i, 0))
    return pl.pallas_call(
        _proj0_kernel,
        grid=(GRID,),
        in_specs=[row_spec,
                  pl.BlockSpec((BLK, 8), lambda i: (i, 0)),
                  pl.BlockSpec((D, D), lambda i: (0, 0)),
                  pl.BlockSpec((D, D), lambda i: (0, 0)),
                  pl.BlockSpec((D, D), lambda i: (0, 0))],
        out_specs=[row_spec, row_spec, row_spec],
        out_shape=[jax.ShapeDtypeStruct((N_PAD, D), jnp.float32)] * 3,
    )(x_pad, scales, w0l, w0d, w0r)


def _combine_proj_kernel(p_ref, sc_ref, bsum_ref, wl_ref, wd_ref, wr_ref,
                         yl_ref, yd_ref, yr_ref):
    i = pl.program_id(0)
    s = sc_ref[...]
    p = p_ref[...]
    acc = (p[0, 0] + p[1, 0]) * s[:, 1][:, None]
    acc += (p[0, 1] + p[1, 1]) * s[:, 3][:, None]
    acc += (p[0, 2] + p[1, 2]) * s[:, 5][:, None]
    acc += bsum_ref[...]
    rows = i * BLK + lax.broadcasted_iota(jnp.int32, (BLK, 1), 0)
    h = jnp.where(rows < N, jnp.maximum(acc, 0.0), 0.0)
    for w_ref, y_ref, col in ((wl_ref, yl_ref, 0), (wd_ref, yd_ref, 2),
                              (wr_ref, yr_ref, 4)):
        hs = h * s[:, col][:, None]
        y_ref[...] = jnp.dot(hs, w_ref[...],
                             preferred_element_type=jnp.float32,
                             precision=lax.Precision.HIGHEST)


def _run_combine_proj(p, scales, bsum, w1l, w1d, w1r):
    row_spec = pl.BlockSpec((BLK, D), lambda i: (i, 0))
    return pl.pallas_call(
        _combine_proj_kernel,
        grid=(GRID,),
        in_specs=[pl.BlockSpec((2, 3, BLK, D), lambda i: (0, 0, i, 0)),
                  pl.BlockSpec((BLK, 8), lambda i: (i, 0)),
                  pl.BlockSpec((1, D), lambda i: (0, 0)),
                  pl.BlockSpec((D, D), lambda i: (0, 0)),
                  pl.BlockSpec((D, D), lambda i: (0, 0)),
                  pl.BlockSpec((D, D), lambda i: (0, 0))],
        out_specs=[row_spec, row_spec, row_spec],
        out_shape=[jax.ShapeDtypeStruct((N_PAD, D), jnp.float32)] * 3,
    )(p, scales, bsum, w1l, w1d, w1r)


def _readout_kernel(p_ref, sc_ref, bsum_ref, out_ref):
    i = pl.program_id(0)
    s = sc_ref[...]
    p = p_ref[...]
    acc = (p[0, 0] + p[1, 0]) * s[:, 1][:, None]
    acc += (p[0, 1] + p[1, 1]) * s[:, 3][:, None]
    acc += (p[0, 2] + p[1, 2]) * s[:, 5][:, None]
    acc += bsum_ref[...]
    rows = i * BLK + lax.broadcasted_iota(jnp.int32, (BLK, 1), 0)
    h = jnp.where(rows < N, jnp.maximum(acc, 0.0), 0.0)
    part = jnp.sum(h, axis=0, keepdims=True) * (1.0 / N)

    @pl.when(i == 0)
    def _():
        out_ref[...] = part

    @pl.when(i > 0)
    def _():
        out_ref[...] += part


def _run_readout(p, scales, bsum):
    return pl.pallas_call(
        _readout_kernel,
        grid=(GRID,),
        in_specs=[pl.BlockSpec((2, 3, BLK, D), lambda i: (0, 0, i, 0)),
                  pl.BlockSpec((BLK, 8), lambda i: (i, 0)),
                  pl.BlockSpec((1, D), lambda i: (0, 0))],
        out_specs=pl.BlockSpec((1, D), lambda i: (0, 0)),
        out_shape=jax.ShapeDtypeStruct((1, D), jnp.float32),
    )(p, scales, bsum)


def kernel(x, edge_index_loop, edge_index_dep, edge_index_rdep,
           W0_loop, b0_loop, W0_dep, b0_dep, W0_rdep, b0_rdep,
           W1_loop, b1_loop, W1_dep, b1_dep, W1_rdep, b1_rdep):
    src_all, dst_all = _stage_edges(edge_index_loop, edge_index_dep,
                                    edge_index_rdep)
    src_a, dst_a = _stage_edges_asym(edge_index_loop, edge_index_dep,
                                     edge_index_rdep)
    x_pad = jnp.pad(x, ((0, N_PAD - N), (0, 0)))

    degp = _run_deg(src_all, dst_all)
    scales = _run_scale(degp)

    b0sum = (b0_loop + b0_dep + b0_rdep).reshape(1, D)
    b1sum = (b1_loop + b1_dep + b1_rdep).reshape(1, D)

    y0l, y0d, y0r = _run_proj0(x_pad, scales, W0_loop, W0_dep, W0_rdep)
    p0 = _run_agg(y0l, y0d, y0r, src_a, dst_a)
    y1l, y1d, y1r = _run_combine_proj(p0, scales, b0sum, W1_loop, W1_dep, W1_rdep)
    p1 = _run_agg(y1l, y1d, y1r, src_a, dst_a)
    return _run_readout(p1, scales, b1sum)


# speedup vs baseline: 1.4504x; 1.0017x over previous
import functools
import jax
import jax.numpy as jnp
from jax import lax
from jax.experimental import pallas as pl
from jax.experimental.pallas import tpu as pltpu, tpu_sc as plsc

N = 10000
D = 128
N_PAD = 10240
SINK = 10200
K = 128
NW = 32
ROWS_PER_TILE = N_PAD // 16
BLK = 1024
GRID = N_PAD // BLK

E_LOOP_PAD = 16384
E_DEP_PAD = 163840


CH_LOOP = E_LOOP_PAD // (NW * K)
CH_DEP = E_DEP_PAD // (NW * K)
CH_TOT = CH_LOOP + 2 * CH_DEP
REL_OFF = (2 * CH_DEP, 0, CH_DEP)
REL_CH = (CH_LOOP, CH_DEP, CH_DEP)
NB = 2
NBD = 4


def _pad_edges(ei, e_pad):
    e = ei.shape[1]
    pad = jnp.full((e_pad - e,), SINK, dtype=jnp.int32)
    src = jnp.concatenate([ei[0].astype(jnp.int32), pad])
    dst = jnp.concatenate([ei[1].astype(jnp.int32), pad])
    return src, dst


def _stage_edges(edge_index_loop, edge_index_dep, edge_index_rdep):
    sl, dl = _pad_edges(edge_index_loop, E_LOOP_PAD)
    sd, dd = _pad_edges(edge_index_dep, E_DEP_PAD)
    sr, dr = _pad_edges(edge_index_rdep, E_DEP_PAD)
    def cat(dep, rdep, loop):
        return jnp.concatenate(
            [dep.reshape(NW, CH_DEP, K), rdep.reshape(NW, CH_DEP, K),
             loop.reshape(NW, CH_LOOP, K)], axis=1)
    return cat(sd, sr, sl), cat(dd, dr, dl)


def _deg_kernel(src_all, dst_all, out_hbm, idx_v, ones_v,
                deg_sh, s0, s1, s2, s3):
    cid = lax.axis_index("c")
    sid = lax.axis_index("s")
    wid = cid * 16 + sid
    row0 = sid * ROWS_PER_TILE
    sems = (s0, s1, s2, s3)

    z16 = jnp.zeros((16,), jnp.float32)

    def zinit(i, _):
        for j in range(8):
            ones_v[i, pl.ds(j * 16, 16)] = z16
        return 0
    lax.fori_loop(0, K, zinit, 0, unroll=False)

    for kk in range(ROWS_PER_TILE // K):
        pltpu.sync_copy(ones_v, deg_sh.at[pl.ds(row0 + kk * K, K)])
    plsc.subcore_barrier()

    streams = [(src_all, REL_OFF[0], CH_LOOP), (dst_all, REL_OFF[0], CH_LOOP),
               (src_all, REL_OFF[1], CH_DEP), (dst_all, REL_OFF[1], CH_DEP),
               (src_all, REL_OFF[2], CH_DEP), (dst_all, REL_OFF[2], CH_DEP)]
    for q, (arr, roff, cpw) in enumerate(streams):
        eq = jnp.where(lax.iota(jnp.int32, 16) == q, 1.0, 0.0).astype(jnp.float32)

        def init_body(i, _):
            ones_v[i, pl.ds(0, 16)] = eq
            return 0
        lax.fori_loop(0, K, init_body, 0, unroll=False)

        pltpu.sync_copy(arr.at[wid, pl.ds(roff, cpw)], idx_v.at[pl.ds(0, cpw)])

        for j in range(cpw):
            b = j % NBD
            if j >= NBD:
                pltpu.make_async_copy(ones_v, deg_sh.at[pl.ds(0, K)],
                                      sems[b]).wait()
            pltpu.async_copy(ones_v, deg_sh.at[idx_v.at[j]],
                             sems[b], add=True)
        for b in range(min(NBD, cpw)):
            pltpu.make_async_copy(ones_v, deg_sh.at[pl.ds(0, K)],
                                  sems[b]).wait()
    plsc.subcore_barrier()

    for kk in range(ROWS_PER_TILE // K):
        r0 = row0 + kk * K
        pltpu.sync_copy(deg_sh.at[pl.ds(r0, K)], ones_v)
        pltpu.sync_copy(ones_v, out_hbm.at[cid, pl.ds(r0, K)])


def _run_deg(src_all, dst_all):
    k = pl.kernel(
        _deg_kernel,
        out_type=jax.ShapeDtypeStruct((2, N_PAD, D), jnp.float32),
        mesh=plsc.VectorSubcoreMesh(core_axis_name="c", subcore_axis_name="s"),
        scratch_types=[
            pltpu.VMEM((CH_DEP, K), jnp.int32),
            pltpu.VMEM((K, D), jnp.float32),
            pltpu.VMEM_SHARED((N_PAD, D), jnp.float32),
            pltpu.SemaphoreType.DMA,
            pltpu.SemaphoreType.DMA,
            pltpu.SemaphoreType.DMA,
            pltpu.SemaphoreType.DMA,
        ],
    )
    return k(src_all, dst_all)


FAST_CID = 0
FCH = (8, 56, 56)
SCH = (0, 24, 24)
CH_ROW = 136
REL_OFF_A = (128, 0, 64)
SEG = 32


def _stage_edges_asym(edge_index_loop, edge_index_dep, edge_index_rdep):
    sl, dl = _pad_edges(edge_index_loop, E_LOOP_PAD)
    sd, dd = _pad_edges(edge_index_dep, E_DEP_PAD)
    sr, dr = _pad_edges(edge_index_rdep, E_DEP_PAD)

    def split(arr, fc, sc, slots):
        c = arr.reshape(-1, K)
        fast = c[:16 * fc].reshape(16, fc, K)
        slow = c[16 * fc:].reshape(16, sc, K) if sc else jnp.zeros(
            (16, 0, K), jnp.int32)
        fast = jnp.pad(fast, ((0, 0), (0, slots - fc), (0, 0)))
        slow = jnp.pad(slow, ((0, 0), (0, slots - sc), (0, 0)))
        pair = (fast, slow) if FAST_CID == 0 else (slow, fast)
        return jnp.concatenate(pair, axis=0)

    def lay(lp, dp, rd):
        return jnp.concatenate(
            [split(dp, FCH[1], SCH[1], 64), split(rd, FCH[2], SCH[2], 64),
             split(lp, FCH[0], SCH[0], 8)], axis=1)
    return lay(sl, sd, sr), lay(dl, dd, dr)


def _agg_kernel(yl, yd, yr, src_a, dst_a, out_hbm,
                idxs_v, idxd_v, rb0, rb1, acc_sh, g0, g1, s0, s1):
    cid = lax.axis_index("c")
    sid = lax.axis_index("s")
    wid = cid * 16 + sid
    row0 = sid * ROWS_PER_TILE
    rings = (rb0, rb1)
    gsems = (g0, g1)
    ssems = (s0, s1)

    z16 = jnp.zeros((16,), jnp.float32)

    def chunk(ytab, j, b):
        pltpu.async_copy(ytab.at[idxs_v.at[j]], rings[b], gsems[b])
        pltpu.make_async_copy(ytab.at[pl.ds(0, K)], rings[b], gsems[b]).wait()
        pltpu.async_copy(rings[b], acc_sh.at[idxd_v.at[j]], ssems[b], add=True)

    def pipeline(ytab, roff, cpw):
        for seg0 in range(0, cpw, SEG):
            seg = min(SEG, cpw - seg0)
            pltpu.sync_copy(src_a.at[wid, pl.ds(roff + seg0, seg)],
                            idxs_v.at[pl.ds(0, seg)])
            pltpu.sync_copy(dst_a.at[wid, pl.ds(roff + seg0, seg)],
                            idxd_v.at[pl.ds(0, seg)])
            for b in range(2):
                chunk(ytab, b, b)

            def grp(g, _):
                for b in range(2):
                    pltpu.make_async_copy(rings[b], acc_sh.at[pl.ds(0, K)],
                                          ssems[b]).wait()
                    chunk(ytab, g * 2 + b, b)
                return 0
            lax.fori_loop(1, seg // 2, grp, 0, unroll=False)
            for b in range(2):
                pltpu.make_async_copy(rings[b], acc_sh.at[pl.ds(0, K)],
                                      ssems[b]).wait()

    rels = [(yl, 0), (yd, 1), (yr, 2)]
    for r, (ytab, ri) in enumerate(rels):
        def zinit(i, _):
            for j in range(8):
                rb0[i, pl.ds(j * 16, 16)] = z16
            return 0
        lax.fori_loop(0, K, zinit, 0, unroll=False)
        for kk in range(ROWS_PER_TILE // K):
            pltpu.sync_copy(rb0, acc_sh.at[pl.ds(row0 + kk * K, K)])
        plsc.subcore_barrier()

        roff = REL_OFF_A[ri]
        if SCH[ri]:
            pipeline(ytab, roff, SCH[ri])
        if FCH[ri] > SCH[ri]:
            @pl.when(cid == FAST_CID)
            def _():
                pipeline(ytab, roff + SCH[ri], FCH[ri] - SCH[ri])
        plsc.subcore_barrier()

        for kk in range(ROWS_PER_TILE // K):
            r0 = row0 + kk * K
            pltpu.sync_copy(acc_sh.at[pl.ds(r0, K)], rb0)
            pltpu.sync_copy(rb0, out_hbm.at[cid, r, pl.ds(r0, K)])


def _run_agg(yl, yd, yr, src_a, dst_a):
    k = pl.kernel(
        _agg_kernel,
        out_type=jax.ShapeDtypeStruct((2, 3, N_PAD, D), jnp.float32),
        mesh=plsc.VectorSubcoreMesh(core_axis_name="c", subcore_axis_name="s"),
        scratch_types=[
            pltpu.VMEM((SEG, K), jnp.int32),
            pltpu.VMEM((SEG, K), jnp.int32),
            pltpu.VMEM((K, D), jnp.float32),
            pltpu.VMEM((K, D), jnp.float32),
            pltpu.VMEM_SHARED((N_PAD, D), jnp.float32),
            pltpu.SemaphoreType.DMA,
            pltpu.SemaphoreType.DMA,
            pltpu.SemaphoreType.DMA,
            pltpu.SemaphoreType.DMA,
        ],
    )
    return k(yl, yd, yr, src_a, dst_a)


def _scale_kernel(degp_ref, out_ref):
    p = degp_ref[...]
    deg = (p[0] + p[1])[:, 0:8]
    out_ref[...] = lax.rsqrt(jnp.maximum(deg, 1.0))


def _run_scale(degp):
    return pl.pallas_call(
        _scale_kernel,
        grid=(GRID,),
        in_specs=[pl.BlockSpec((2, BLK, D), lambda i: (0, i, 0))],
        out_specs=pl.BlockSpec((BLK, 8), lambda i: (i, 0)),
        out_shape=jax.ShapeDtypeStruct((N_PAD, 8), jnp.float32),
    )(degp)


def _proj0_kernel(x_ref, sc_ref, wl_ref, wd_ref, wr_ref, yl_ref, yd_ref, yr_ref):
    x = x_ref[...]
    s = sc_ref[...]
    for w_ref, y_ref, col in ((wl_ref, yl_ref, 0), (wd_ref, yd_ref, 2),
                              (wr_ref, yr_ref, 4)):
        xs = x * s[:, col][:, None]
        y_ref[...] = jnp.dot(xs, w_ref[...],
                             preferred_element_type=jnp.float32,
                             precision=lax.Precision.HIGHEST)


def _run_proj0(x_pad, scales, w0l, w0d, w0r):
    row_spec = pl.BlockSpec((BLK, D), lambda i: (i, 0))
    return pl.pallas_call(
        _proj0_kernel,
        grid=(GRID,),
        in_specs=[row_spec,
                  pl.BlockSpec((BLK, 8), lambda i: (i, 0)),
                  pl.BlockSpec((D, D), lambda i: (0, 0)),
                  pl.BlockSpec((D, D), lambda i: (0, 0)),
                  pl.BlockSpec((D, D), lambda i: (0, 0))],
        out_specs=[row_spec, row_spec, row_spec],
        out_shape=[jax.ShapeDtypeStruct((N_PAD, D), jnp.float32)] * 3,
    )(x_pad, scales, w0l, w0d, w0r)


def _combine_proj_kernel(p_ref, sc_ref, bsum_ref, wl_ref, wd_ref, wr_ref,
                         yl_ref, yd_ref, yr_ref):
    i = pl.program_id(0)
    s = sc_ref[...]
    p = p_ref[...]
    acc = (p[0, 0] + p[1, 0]) * s[:, 1][:, None]
    acc += (p[0, 1] + p[1, 1]) * s[:, 3][:, None]
    acc += (p[0, 2] + p[1, 2]) * s[:, 5][:, None]
    acc += bsum_ref[...]
    rows = i * BLK + lax.broadcasted_iota(jnp.int32, (BLK, 1), 0)
    h = jnp.where(rows < N, jnp.maximum(acc, 0.0), 0.0)
    for w_ref, y_ref, col in ((wl_ref, yl_ref, 0), (wd_ref, yd_ref, 2),
                              (wr_ref, yr_ref, 4)):
        hs = h * s[:, col][:, None]
        y_ref[...] = jnp.dot(hs, w_ref[...],
                             preferred_element_type=jnp.float32,
                             precision=lax.Precision.HIGHEST)


def _run_combine_proj(p, scales, bsum, w1l, w1d, w1r):
    row_spec = pl.BlockSpec((BLK, D), lambda i: (i, 0))
    return pl.pallas_call(
        _combine_proj_kernel,
        grid=(GRID,),
        in_specs=[pl.BlockSpec((2, 3, BLK, D), lambda i: (0, 0, i, 0)),
                  pl.BlockSpec((BLK, 8), lambda i: (i, 0)),
                  pl.BlockSpec((1, D), lambda i: (0, 0)),
                  pl.BlockSpec((D, D), lambda i: (0, 0)),
                  pl.BlockSpec((D, D), lambda i: (0, 0)),
                  pl.BlockSpec((D, D), lambda i: (0, 0))],
        out_specs=[row_spec, row_spec, row_spec],
        out_shape=[jax.ShapeDtypeStruct((N_PAD, D), jnp.float32)] * 3,
    )(p, scales, bsum, w1l, w1d, w1r)


def _readout_kernel(p_ref, sc_ref, bsum_ref, out_ref):
    i = pl.program_id(0)
    s = sc_ref[...]
    p = p_ref[...]
    acc = (p[0, 0] + p[1, 0]) * s[:, 1][:, None]
    acc += (p[0, 1] + p[1, 1]) * s[:, 3][:, None]
    acc += (p[0, 2] + p[1, 2]) * s[:, 5][:, None]
    acc += bsum_ref[...]
    rows = i * BLK + lax.broadcasted_iota(jnp.int32, (BLK, 1), 0)
    h = jnp.where(rows < N, jnp.maximum(acc, 0.0), 0.0)
    part = jnp.sum(h, axis=0, keepdims=True) * (1.0 / N)

    @pl.when(i == 0)
    def _():
        out_ref[...] = part

    @pl.when(i > 0)
    def _():
        out_ref[...] += part


def _run_readout(p, scales, bsum):
    return pl.pallas_call(
        _readout_kernel,
        grid=(GRID,),
        in_specs=[pl.BlockSpec((2, 3, BLK, D), lambda i: (0, 0, i, 0)),
                  pl.BlockSpec((BLK, 8), lambda i: (i, 0)),
                  pl.BlockSpec((1, D), lambda i: (0, 0))],
        out_specs=pl.BlockSpec((1, D), lambda i: (0, 0)),
        out_shape=jax.ShapeDtypeStruct((1, D), jnp.float32),
    )(p, scales, bsum)


def kernel(x, edge_index_loop, edge_index_dep, edge_index_rdep,
           W0_loop, b0_loop, W0_dep, b0_dep, W0_rdep, b0_rdep,
           W1_loop, b1_loop, W1_dep, b1_dep, W1_rdep, b1_rdep):
    src_all, dst_all = _stage_edges(edge_index_loop, edge_index_dep,
                                    edge_index_rdep)
    src_a, dst_a = _stage_edges_asym(edge_index_loop, edge_index_dep,
                                     edge_index_rdep)
    x_pad = jnp.pad(x, ((0, N_PAD - N), (0, 0)))

    degp = _run_deg(src_all, dst_all)
    scales = _run_scale(degp)

    b0sum = (b0_loop + b0_dep + b0_rdep).reshape(1, D)
    b1sum = (b1_loop + b1_dep + b1_rdep).reshape(1, D)

    y0l, y0d, y0r = _run_proj0(x_pad, scales, W0_loop, W0_dep, W0_rdep)
    p0 = _run_agg(y0l, y0d, y0r, src_a, dst_a)
    y1l, y1d, y1r = _run_combine_proj(p0, scales, b0sum, W1_loop, W1_dep, W1_rdep)
    p1 = _run_agg(y1l, y1d, y1r, src_a, dst_a)
    return _run_readout(p1, scales, b1sum)


# speedup vs baseline: 1.4527x; 1.0016x over previous
import functools
import jax
import jax.numpy as jnp
from jax import lax
from jax.experimental import pallas as pl
from jax.experimental.pallas import tpu as pltpu, tpu_sc as plsc

N = 10000
D = 128
N_PAD = 10240
SINK = 10200
K = 128
NW = 32
ROWS_PER_TILE = N_PAD // 16
BLK = 1024
GRID = N_PAD // BLK

E_LOOP_PAD = 16384
E_DEP_PAD = 163840


CH_LOOP = E_LOOP_PAD // (NW * K)
CH_DEP = E_DEP_PAD // (NW * K)
CH_TOT = CH_LOOP + 2 * CH_DEP
REL_OFF = (2 * CH_DEP, 0, CH_DEP)
REL_CH = (CH_LOOP, CH_DEP, CH_DEP)
NB = 2
NBD = 4


def _pad_edges(ei, e_pad):
    e = ei.shape[1]
    pad = jnp.full((e_pad - e,), SINK, dtype=jnp.int32)
    src = jnp.concatenate([ei[0].astype(jnp.int32), pad])
    dst = jnp.concatenate([ei[1].astype(jnp.int32), pad])
    return src, dst


def _stage_edges(edge_index_loop, edge_index_dep, edge_index_rdep):
    sl, dl = _pad_edges(edge_index_loop, E_LOOP_PAD)
    sd, dd = _pad_edges(edge_index_dep, E_DEP_PAD)
    sr, dr = _pad_edges(edge_index_rdep, E_DEP_PAD)
    def cat(dep, rdep, loop):
        return jnp.concatenate(
            [dep.reshape(NW, CH_DEP, K), rdep.reshape(NW, CH_DEP, K),
             loop.reshape(NW, CH_LOOP, K)], axis=1)
    return cat(sd, sr, sl), cat(dd, dr, dl)


def _deg_kernel(src_all, dst_all, out_hbm, idx_v, ones_v,
                deg_sh, s0, s1, s2, s3):
    cid = lax.axis_index("c")
    sid = lax.axis_index("s")
    wid = cid * 16 + sid
    row0 = sid * ROWS_PER_TILE
    sems = (s0, s1, s2, s3)

    z16 = jnp.zeros((16,), jnp.float32)

    def zinit(i, _):
        for j in range(8):
            ones_v[i, pl.ds(j * 16, 16)] = z16
        return 0
    lax.fori_loop(0, K, zinit, 0, unroll=False)

    for kk in range(ROWS_PER_TILE // K):
        pltpu.async_copy(ones_v, deg_sh.at[pl.ds(row0 + kk * K, K)], s0)
    for kk in range(ROWS_PER_TILE // K):
        pltpu.make_async_copy(ones_v, deg_sh.at[pl.ds(row0, K)], s0).wait()
    plsc.subcore_barrier()

    streams = [(src_all, REL_OFF[0], CH_LOOP), (dst_all, REL_OFF[0], CH_LOOP),
               (src_all, REL_OFF[1], CH_DEP), (dst_all, REL_OFF[1], CH_DEP),
               (src_all, REL_OFF[2], CH_DEP), (dst_all, REL_OFF[2], CH_DEP)]
    for q, (arr, roff, cpw) in enumerate(streams):
        eq = jnp.where(lax.iota(jnp.int32, 16) == q, 1.0, 0.0).astype(jnp.float32)

        def init_body(i, _):
            ones_v[i, pl.ds(0, 16)] = eq
            return 0
        lax.fori_loop(0, K, init_body, 0, unroll=False)

        pltpu.sync_copy(arr.at[wid, pl.ds(roff, cpw)], idx_v.at[pl.ds(0, cpw)])

        for j in range(cpw):
            b = j % NBD
            if j >= NBD:
                pltpu.make_async_copy(ones_v, deg_sh.at[pl.ds(0, K)],
                                      sems[b]).wait()
            pltpu.async_copy(ones_v, deg_sh.at[idx_v.at[j]],
                             sems[b], add=True)
        for b in range(min(NBD, cpw)):
            pltpu.make_async_copy(ones_v, deg_sh.at[pl.ds(0, K)],
                                  sems[b]).wait()
    plsc.subcore_barrier()

    for kk in range(ROWS_PER_TILE // K):
        r0 = row0 + kk * K
        pltpu.async_copy(deg_sh.at[pl.ds(r0, K)],
                         out_hbm.at[cid, pl.ds(r0, K)], s1)
    for kk in range(ROWS_PER_TILE // K):
        pltpu.make_async_copy(deg_sh.at[pl.ds(row0, K)],
                              out_hbm.at[cid, pl.ds(row0, K)], s1).wait()


def _run_deg(src_all, dst_all):
    k = pl.kernel(
        _deg_kernel,
        out_type=jax.ShapeDtypeStruct((2, N_PAD, D), jnp.float32),
        mesh=plsc.VectorSubcoreMesh(core_axis_name="c", subcore_axis_name="s"),
        scratch_types=[
            pltpu.VMEM((CH_DEP, K), jnp.int32),
            pltpu.VMEM((K, D), jnp.float32),
            pltpu.VMEM_SHARED((N_PAD, D), jnp.float32),
            pltpu.SemaphoreType.DMA,
            pltpu.SemaphoreType.DMA,
            pltpu.SemaphoreType.DMA,
            pltpu.SemaphoreType.DMA,
        ],
    )
    return k(src_all, dst_all)


FAST_CID = 0
FCH = (8, 56, 56)
SCH = (0, 24, 24)
CH_ROW = 136
REL_OFF_A = (128, 0, 64)
SEG = 32


def _stage_edges_asym(edge_index_loop, edge_index_dep, edge_index_rdep):
    sl, dl = _pad_edges(edge_index_loop, E_LOOP_PAD)
    sd, dd = _pad_edges(edge_index_dep, E_DEP_PAD)
    sr, dr = _pad_edges(edge_index_rdep, E_DEP_PAD)

    def split(arr, fc, sc, slots):
        c = arr.reshape(-1, K)
        fast = c[:16 * fc].reshape(16, fc, K)
        slow = c[16 * fc:].reshape(16, sc, K) if sc else jnp.zeros(
            (16, 0, K), jnp.int32)
        fast = jnp.pad(fast, ((0, 0), (0, slots - fc), (0, 0)))
        slow = jnp.pad(slow, ((0, 0), (0, slots - sc), (0, 0)))
        pair = (fast, slow) if FAST_CID == 0 else (slow, fast)
        return jnp.concatenate(pair, axis=0)

    def lay(lp, dp, rd):
        return jnp.concatenate(
            [split(dp, FCH[1], SCH[1], 64), split(rd, FCH[2], SCH[2], 64),
             split(lp, FCH[0], SCH[0], 8)], axis=1)
    return lay(sl, sd, sr), lay(dl, dd, dr)


def _agg_kernel(yl, yd, yr, src_a, dst_a, out_hbm,
                idxs_v, idxd_v, rb0, rb1, acc_sh, g0, g1, s0, s1):
    cid = lax.axis_index("c")
    sid = lax.axis_index("s")
    wid = cid * 16 + sid
    row0 = sid * ROWS_PER_TILE
    rings = (rb0, rb1)
    gsems = (g0, g1)
    ssems = (s0, s1)

    z16 = jnp.zeros((16,), jnp.float32)

    def chunk(ytab, j, b):
        pltpu.async_copy(ytab.at[idxs_v.at[j]], rings[b], gsems[b])
        pltpu.make_async_copy(ytab.at[pl.ds(0, K)], rings[b], gsems[b]).wait()
        pltpu.async_copy(rings[b], acc_sh.at[idxd_v.at[j]], ssems[b], add=True)

    def pipeline(ytab, roff, cpw):
        for seg0 in range(0, cpw, SEG):
            seg = min(SEG, cpw - seg0)
            pltpu.sync_copy(src_a.at[wid, pl.ds(roff + seg0, seg)],
                            idxs_v.at[pl.ds(0, seg)])
            pltpu.sync_copy(dst_a.at[wid, pl.ds(roff + seg0, seg)],
                            idxd_v.at[pl.ds(0, seg)])
            for b in range(2):
                chunk(ytab, b, b)

            def grp(g, _):
                for b in range(2):
                    pltpu.make_async_copy(rings[b], acc_sh.at[pl.ds(0, K)],
                                          ssems[b]).wait()
                    chunk(ytab, g * 2 + b, b)
                return 0
            lax.fori_loop(1, seg // 2, grp, 0, unroll=False)
            for b in range(2):
                pltpu.make_async_copy(rings[b], acc_sh.at[pl.ds(0, K)],
                                      ssems[b]).wait()

    rels = [(yl, 0), (yd, 1), (yr, 2)]
    for r, (ytab, ri) in enumerate(rels):
        def zinit(i, _):
            for j in range(8):
                rb0[i, pl.ds(j * 16, 16)] = z16
            return 0
        lax.fori_loop(0, K, zinit, 0, unroll=False)
        for kk in range(ROWS_PER_TILE // K):
            pltpu.async_copy(rb0, acc_sh.at[pl.ds(row0 + kk * K, K)], g0)
        for kk in range(ROWS_PER_TILE // K):
            pltpu.make_async_copy(rb0, acc_sh.at[pl.ds(row0, K)], g0).wait()
        plsc.subcore_barrier()

        roff = REL_OFF_A[ri]
        if SCH[ri]:
            pipeline(ytab, roff, SCH[ri])
        if FCH[ri] > SCH[ri]:
            @pl.when(cid == FAST_CID)
            def _():
                pipeline(ytab, roff + SCH[ri], FCH[ri] - SCH[ri])
        plsc.subcore_barrier()

        for kk in range(ROWS_PER_TILE // K):
            r0 = row0 + kk * K
            pltpu.async_copy(acc_sh.at[pl.ds(r0, K)],
                             out_hbm.at[cid, r, pl.ds(r0, K)], g1)
        for kk in range(ROWS_PER_TILE // K):
            pltpu.make_async_copy(acc_sh.at[pl.ds(row0, K)],
                                  out_hbm.at[cid, r, pl.ds(row0, K)], g1).wait()


def _run_agg(yl, yd, yr, src_a, dst_a):
    k = pl.kernel(
        _agg_kernel,
        out_type=jax.ShapeDtypeStruct((2, 3, N_PAD, D), jnp.float32),
        mesh=plsc.VectorSubcoreMesh(core_axis_name="c", subcore_axis_name="s"),
        scratch_types=[
            pltpu.VMEM((SEG, K), jnp.int32),
            pltpu.VMEM((SEG, K), jnp.int32),
            pltpu.VMEM((K, D), jnp.float32),
            pltpu.VMEM((K, D), jnp.float32),
            pltpu.VMEM_SHARED((N_PAD, D), jnp.float32),
            pltpu.SemaphoreType.DMA,
            pltpu.SemaphoreType.DMA,
            pltpu.SemaphoreType.DMA,
            pltpu.SemaphoreType.DMA,
        ],
    )
    return k(yl, yd, yr, src_a, dst_a)


def _scale_kernel(degp_ref, out_ref):
    p = degp_ref[...]
    deg = (p[0] + p[1])[:, 0:8]
    out_ref[...] = lax.rsqrt(jnp.maximum(deg, 1.0))


def _run_scale(degp):
    return pl.pallas_call(
        _scale_kernel,
        grid=(GRID,),
        in_specs=[pl.BlockSpec((2, BLK, D), lambda i: (0, i, 0))],
        out_specs=pl.BlockSpec((BLK, 8), lambda i: (i, 0)),
        out_shape=jax.ShapeDtypeStruct((N_PAD, 8), jnp.float32),
    )(degp)


def _proj0_kernel(x_ref, sc_ref, wl_ref, wd_ref, wr_ref, yl_ref, yd_ref, yr_ref):
    x = x_ref[...]
    s = sc_ref[...]
    for w_ref, y_ref, col in ((wl_ref, yl_ref, 0), (wd_ref, yd_ref, 2),
                              (wr_ref, yr_ref, 4)):
        xs = x * s[:, col][:, None]
        y_ref[...] = jnp.dot(xs, w_ref[...],
                             preferred_element_type=jnp.float32,
                             precision=lax.Precision.HIGHEST)


def _run_proj0(x_pad, scales, w0l, w0d, w0r):
    row_spec = pl.BlockSpec((BLK, D), lambda i: (i, 0))
    return pl.pallas_call(
        _proj0_kernel,
        grid=(GRID,),
        in_specs=[row_spec,
                  pl.BlockSpec((BLK, 8), lambda i: (i, 0)),
                  pl.BlockSpec((D, D), lambda i: (0, 0)),
                  pl.BlockSpec((D, D), lambda i: (0, 0)),
                  pl.BlockSpec((D, D), lambda i: (0, 0))],
        out_specs=[row_spec, row_spec, row_spec],
        out_shape=[jax.ShapeDtypeStruct((N_PAD, D), jnp.float32)] * 3,
    )(x_pad, scales, w0l, w0d, w0r)


def _combine_proj_kernel(p_ref, sc_ref, bsum_ref, wl_ref, wd_ref, wr_ref,
                         yl_ref, yd_ref, yr_ref):
    i = pl.program_id(0)
    s = sc_ref[...]
    p = p_ref[...]
    acc = (p[0, 0] + p[1, 0]) * s[:, 1][:, None]
    acc += (p[0, 1] + p[1, 1]) * s[:, 3][:, None]
    acc += (p[0, 2] + p[1, 2]) * s[:, 5][:, None]
    acc += bsum_ref[...]
    rows = i * BLK + lax.broadcasted_iota(jnp.int32, (BLK, 1), 0)
    h = jnp.where(rows < N, jnp.maximum(acc, 0.0), 0.0)
    for w_ref, y_ref, col in ((wl_ref, yl_ref, 0), (wd_ref, yd_ref, 2),
                              (wr_ref, yr_ref, 4)):
        hs = h * s[:, col][:, None]
        y_ref[...] = jnp.dot(hs, w_ref[...],
                             preferred_element_type=jnp.float32,
                             precision=lax.Precision.HIGHEST)


def _run_combine_proj(p, scales, bsum, w1l, w1d, w1r):
    row_spec = pl.BlockSpec((BLK, D), lambda i: (i, 0))
    return pl.pallas_call(
        _combine_proj_kernel,
        grid=(GRID,),
        in_specs=[pl.BlockSpec((2, 3, BLK, D), lambda i: (0, 0, i, 0)),
                  pl.BlockSpec((BLK, 8), lambda i: (i, 0)),
                  pl.BlockSpec((1, D), lambda i: (0, 0)),
                  pl.BlockSpec((D, D), lambda i: (0, 0)),
                  pl.BlockSpec((D, D), lambda i: (0, 0)),
                  pl.BlockSpec((D, D), lambda i: (0, 0))],
        out_specs=[row_spec, row_spec, row_spec],
        out_shape=[jax.ShapeDtypeStruct((N_PAD, D), jnp.float32)] * 3,
    )(p, scales, bsum, w1l, w1d, w1r)


def _readout_kernel(p_ref, sc_ref, bsum_ref, out_ref):
    i = pl.program_id(0)
    s = sc_ref[...]
    p = p_ref[...]
    acc = (p[0, 0] + p[1, 0]) * s[:, 1][:, None]
    acc += (p[0, 1] + p[1, 1]) * s[:, 3][:, None]
    acc += (p[0, 2] + p[1, 2]) * s[:, 5][:, None]
    acc += bsum_ref[...]
    rows = i * BLK + lax.broadcasted_iota(jnp.int32, (BLK, 1), 0)
    h = jnp.where(rows < N, jnp.maximum(acc, 0.0), 0.0)
    part = jnp.sum(h, axis=0, keepdims=True) * (1.0 / N)

    @pl.when(i == 0)
    def _():
        out_ref[...] = part

    @pl.when(i > 0)
    def _():
        out_ref[...] += part


def _run_readout(p, scales, bsum):
    return pl.pallas_call(
        _readout_kernel,
        grid=(GRID,),
        in_specs=[pl.BlockSpec((2, 3, BLK, D), lambda i: (0, 0, i, 0)),
                  pl.BlockSpec((BLK, 8), lambda i: (i, 0)),
                  pl.BlockSpec((1, D), lambda i: (0, 0))],
        out_specs=pl.BlockSpec((1, D), lambda i: (0, 0)),
        out_shape=jax.ShapeDtypeStruct((1, D), jnp.float32),
    )(p, scales, bsum)


def kernel(x, edge_index_loop, edge_index_dep, edge_index_rdep,
           W0_loop, b0_loop, W0_dep, b0_dep, W0_rdep, b0_rdep,
           W1_loop, b1_loop, W1_dep, b1_dep, W1_rdep, b1_rdep):
    src_all, dst_all = _stage_edges(edge_index_loop, edge_index_dep,
                                    edge_index_rdep)
    src_a, dst_a = _stage_edges_asym(edge_index_loop, edge_index_dep,
                                     edge_index_rdep)
    x_pad = jnp.pad(x, ((0, N_PAD - N), (0, 0)))

    degp = _run_deg(src_all, dst_all)
    scales = _run_scale(degp)

    b0sum = (b0_loop + b0_dep + b0_rdep).reshape(1, D)
    b1sum = (b1_loop + b1_dep + b1_rdep).reshape(1, D)

    y0l, y0d, y0r = _run_proj0(x_pad, scales, W0_loop, W0_dep, W0_rdep)
    p0 = _run_agg(y0l, y0d, y0r, src_a, dst_a)
    y1l, y1d, y1r = _run_combine_proj(p0, scales, b0sum, W1_loop, W1_dep, W1_rdep)
    p1 = _run_agg(y1l, y1d, y1r, src_a, dst_a)
    return _run_readout(p1, scales, b1sum)
